# baseline (speedup 1.0000x reference)
import functools

import jax
import jax.numpy as jnp
from jax import lax
from jax.experimental import pallas as pl
from jax.experimental.pallas import tpu as pltpu
from jax.experimental.pallas import tpu_sc as plsc

N_NODES = 10000
N_ACC = 10240
N_EDGES = 320000
NC, NS = 2, 16
CHUNK = 128
CHUNKS_PER_TILE = 80
E_PAD = NC * NS * CHUNKS_PER_TILE * CHUNK
E_ROWS = E_PAD // CHUNK
ROWS_PER_TILE = N_ACC // NS
MAXPAD = 4


def _sc_mesh():
    return plsc.VectorSubcoreMesh(core_axis_name="c", subcore_axis_name="s")


_SC_PARAMS = pltpu.CompilerParams(use_tc_tiling_on_sc=False)


def _degree_body(dst_hbm, out_hbm, dst_v, ones_v, deg_sh, sem):
    c = lax.axis_index("c")
    s = lax.axis_index("s")
    w = c * NS + s
    init = jnp.where(c == 0, 1.0, 0.0).astype(jnp.float32)

    for k in range(CHUNK // 16):
        ones_v[pl.ds(16 * k, 16)] = jnp.zeros((16,), jnp.float32) + init

    def zinit(t, _):
        pltpu.sync_copy(ones_v, deg_sh.at[pl.ds(ROWS_PER_TILE * s + CHUNK * t, CHUNK)])
        return 0

    lax.fori_loop(0, ROWS_PER_TILE // CHUNK, zinit, 0)
    for k in range(CHUNK // 16):
        ones_v[pl.ds(16 * k, 16)] = jnp.zeros((16,), jnp.float32) + 1.0
    plsc.subcore_barrier()

    pltpu.sync_copy(dst_hbm.at[pl.ds(w * CHUNKS_PER_TILE, CHUNKS_PER_TILE)], dst_v)

    def step(j, _):
        pltpu.sync_copy(ones_v, deg_sh.at[dst_v.at[j]], add=True)
        return 0

    lax.fori_loop(0, CHUNKS_PER_TILE, step, 0)
    plsc.subcore_barrier()
    pltpu.sync_copy(deg_sh.at[pl.ds(ROWS_PER_TILE * s, ROWS_PER_TILE)],
                    out_hbm.at[c, pl.ds(ROWS_PER_TILE * s, ROWS_PER_TILE)])


def _sc_degree(dst2):
    k = functools.partial(
        pl.kernel,
        out_type=jax.ShapeDtypeStruct((NC, N_ACC), jnp.float32),
        mesh=_sc_mesh(),
        scratch_types=[
            pltpu.VMEM((CHUNKS_PER_TILE, CHUNK), jnp.int32),
            pltpu.VMEM((CHUNK,), jnp.float32),
            pltpu.VMEM_SHARED((N_ACC,), jnp.float32),
            pltpu.SemaphoreType.DMA,
        ],
        compiler_params=_SC_PARAMS,
    )(_degree_body)
    return k(dst2)


def _make_agg_body(F, nbuf):
    def body(g_hbm, src_hbm, dst_hbm, out_hbm, *refs):
        src_v, dst_v = refs[0], refs[1]
        rows = refs[2:2 + nbuf]
        acc_sh = refs[2 + nbuf]
        sems = refs[3 + nbuf:]
        rows0 = rows[0]
        c = lax.axis_index("c")
        s = lax.axis_index("s")
        w = c * NS + s

        def zrow(r, _):
            for k in range(F // 16):
                rows0[r, pl.ds(16 * k, 16)] = jnp.zeros((16,), jnp.float32)
            return 0

        lax.fori_loop(0, CHUNK, zrow, 0)

        def zinit(t, _):
            pltpu.sync_copy(
                rows0, acc_sh.at[pl.ds(ROWS_PER_TILE * s + CHUNK * t, CHUNK)])
            return 0

        lax.fori_loop(0, ROWS_PER_TILE // CHUNK, zinit, 0)
        plsc.subcore_barrier()

        pltpu.sync_copy(
            src_hbm.at[pl.ds(w * (CHUNKS_PER_TILE + MAXPAD), CHUNKS_PER_TILE + MAXPAD)],
            src_v)
        pltpu.sync_copy(dst_hbm.at[pl.ds(w * CHUNKS_PER_TILE, CHUNKS_PER_TILE)], dst_v)

        for b in range(nbuf):
            pltpu.async_copy(g_hbm.at[src_v.at[b]], rows[b], sems[b])

        def outer(j, _):
            for b in range(nbuf):
                cur = nbuf * j + b
                pltpu.make_async_copy(
                    g_hbm.at[src_v.at[cur]], rows[b], sems[b]).wait()
                pltpu.sync_copy(rows[b], acc_sh.at[dst_v.at[cur]], add=True)
                pltpu.async_copy(g_hbm.at[src_v.at[cur + nbuf]], rows[b], sems[b])
            return 0

        lax.fori_loop(0, CHUNKS_PER_TILE // nbuf, outer, 0)

        for b in range(nbuf):
            pltpu.make_async_copy(
                g_hbm.at[src_v.at[CHUNKS_PER_TILE + b]], rows[b], sems[b]).wait()

        plsc.subcore_barrier()
        pltpu.sync_copy(acc_sh.at[pl.ds(ROWS_PER_TILE * s, ROWS_PER_TILE)],
                        out_hbm.at[c, pl.ds(ROWS_PER_TILE * s, ROWS_PER_TILE)])

    return body


def _make_agg_body_sync(F):
    def body(g_hbm, src_hbm, dst_hbm, out_hbm, src_v, dst_v, rows, acc_sh, sem):
        c = lax.axis_index("c")
        s = lax.axis_index("s")
        w = c * NS + s

        def zrow(r, _):
            for k in range(F // 16):
                rows[r, pl.ds(16 * k, 16)] = jnp.zeros((16,), jnp.float32)
            return 0

        lax.fori_loop(0, CHUNK, zrow, 0)

        def zinit(t, _):
            pltpu.sync_copy(
                rows, acc_sh.at[pl.ds(ROWS_PER_TILE * s + CHUNK * t, CHUNK)])
            return 0

        lax.fori_loop(0, ROWS_PER_TILE // CHUNK, zinit, 0)
        plsc.subcore_barrier()

        pltpu.sync_copy(
            src_hbm.at[pl.ds(w * (CHUNKS_PER_TILE + MAXPAD), CHUNKS_PER_TILE + MAXPAD)],
            src_v)
        pltpu.sync_copy(dst_hbm.at[pl.ds(w * CHUNKS_PER_TILE, CHUNKS_PER_TILE)], dst_v)

        def step(j, _):
            pltpu.sync_copy(g_hbm.at[src_v.at[j]], rows)
            pltpu.sync_copy(rows, acc_sh.at[dst_v.at[j]], add=True)
            return 0

        lax.fori_loop(0, CHUNKS_PER_TILE, step, 0)
        plsc.subcore_barrier()
        pltpu.sync_copy(acc_sh.at[pl.ds(ROWS_PER_TILE * s, ROWS_PER_TILE)],
                        out_hbm.at[c, pl.ds(ROWS_PER_TILE * s, ROWS_PER_TILE)])

    return body


def _sc_aggregate(g, src3, dst2):
    F = g.shape[1]
    if F >= 128:
        nbuf = 1
        body = _make_agg_body_sync(F)
    else:
        nbuf = 4
        body = _make_agg_body(F, nbuf)
    k = functools.partial(
        pl.kernel,
        out_type=jax.ShapeDtypeStruct((NC, N_ACC, F), jnp.float32),
        mesh=_sc_mesh(),
        scratch_types=(
            [pltpu.VMEM((CHUNKS_PER_TILE + MAXPAD, CHUNK), jnp.int32),
             pltpu.VMEM((CHUNKS_PER_TILE, CHUNK), jnp.int32)]
            + [pltpu.VMEM((CHUNK, F), jnp.float32) for _ in range(nbuf)]
            + [pltpu.VMEM_SHARED((N_ACC, F), jnp.float32)]
            + [pltpu.SemaphoreType.DMA]
            + [pltpu.SemaphoreType.DMA for _ in range(nbuf - 1)]
        ),
        compiler_params=_SC_PARAMS,
    )(body)
    return k(g, src3, dst2)


BN = 2000


def _mm_scale_body(x_ref, w_ref, dv_ref, g_ref):
    h = jnp.dot(x_ref[...], w_ref[...], preferred_element_type=jnp.float32)
    g_ref[...] = h * dv_ref[...]


def _mm_scale(x, W, dinv):
    Fin, Fout = W.shape
    grid = (N_NODES // BN,)
    return pl.pallas_call(
        _mm_scale_body,
        grid=grid,
        in_specs=[
            pl.BlockSpec((BN, Fin), lambda i: (i, 0)),
            pl.BlockSpec((Fin, Fout), lambda i: (0, 0)),
            pl.BlockSpec((BN, 1), lambda i: (i, 0)),
        ],
        out_specs=pl.BlockSpec((BN, Fout), lambda i: (i, 0)),
        out_shape=jax.ShapeDtypeStruct((N_NODES, Fout), jnp.float32),
    )(x, W, dinv)


def _fused_layer_body(acc_ref, g_ref, dv_ref, b_ref, w_ref, out_ref):
    dv = dv_ref[...]
    y = (acc_ref[0] + acc_ref[1] + g_ref[...]) * dv + b_ref[...]
    y = jnp.maximum(y, 0.0)
    out_ref[...] = jnp.dot(y, w_ref[...], preferred_element_type=jnp.float32) * dv


def _fused_layer(acc, g, dinv, b, W):
    Fin, Fout = W.shape
    grid = (N_NODES // BN,)
    return pl.pallas_call(
        _fused_layer_body,
        grid=grid,
        in_specs=[
            pl.BlockSpec((NC, BN, Fin), lambda i: (0, i, 0)),
            pl.BlockSpec((BN, Fin), lambda i: (i, 0)),
            pl.BlockSpec((BN, 1), lambda i: (i, 0)),
            pl.BlockSpec((1, Fin), lambda i: (0, 0)),
            pl.BlockSpec((Fin, Fout), lambda i: (0, 0)),
        ],
        out_specs=pl.BlockSpec((BN, Fout), lambda i: (i, 0)),
        out_shape=jax.ShapeDtypeStruct((N_NODES, Fout), jnp.float32),
    )(acc, g, dinv, b.reshape(1, Fin), W)


def _epilogue_body(acc_ref, g_ref, dv_ref, b_ref, out_ref):
    out_ref[...] = (acc_ref[0] + acc_ref[1] + g_ref[...]) * dv_ref[...] + b_ref[...]


def _epilogue(acc, g, dinv, b):
    F = g.shape[1]
    grid = (N_NODES // BN,)
    return pl.pallas_call(
        _epilogue_body,
        grid=grid,
        in_specs=[
            pl.BlockSpec((NC, BN, F), lambda i: (0, i, 0)),
            pl.BlockSpec((BN, F), lambda i: (i, 0)),
            pl.BlockSpec((BN, 1), lambda i: (i, 0)),
            pl.BlockSpec((1, F), lambda i: (0, 0)),
        ],
        out_specs=pl.BlockSpec((BN, F), lambda i: (i, 0)),
        out_shape=jax.ShapeDtypeStruct((N_NODES, F), jnp.float32),
    )(acc, g, dinv, b.reshape(1, F))


def _final_mm_body(flat_ref, we_ref, be_ref, z_ref):
    k = pl.program_id(0)

    @pl.when(k == 0)
    def _init():
        z_ref[...] = be_ref[...]

    z_ref[...] += jnp.dot(flat_ref[...], we_ref[...],
                          preferred_element_type=jnp.float32)


def _final_matmul(flat, We, be):
    K = flat.shape[1]
    BK = 16000
    grid = (K // BK,)
    return pl.pallas_call(
        _final_mm_body,
        grid=grid,
        in_specs=[
            pl.BlockSpec((1, BK), lambda k: (0, k)),
            pl.BlockSpec((BK, 128), lambda k: (k, 0)),
            pl.BlockSpec((1, 128), lambda k: (0, 0)),
        ],
        out_specs=pl.BlockSpec((1, 128), lambda k: (0, 0)),
        out_shape=jax.ShapeDtypeStruct((1, 128), jnp.float32),
    )(flat, We, be.reshape(1, 128))


def kernel(x, edge_index, W1, b1, W2, b2, W3, b3, We, be):
    src = edge_index[0].astype(jnp.int32)
    dst = edge_index[1].astype(jnp.int32)
    extra = E_PAD - N_EDGES
    pad_src = jnp.arange(extra, dtype=jnp.int32) % N_NODES
    pad_dst = N_NODES + jnp.arange(extra, dtype=jnp.int32) % (N_ACC - N_NODES)
    src2 = jnp.concatenate([src, pad_src]).reshape(E_ROWS, CHUNK)
    dst2 = jnp.concatenate([dst, pad_dst]).reshape(E_ROWS, CHUNK)
    NW = NC * NS
    src3 = jnp.concatenate(
        [src2.reshape(NW, CHUNKS_PER_TILE, CHUNK),
         jnp.zeros((NW, MAXPAD, CHUNK), jnp.int32)], axis=1,
    ).reshape(NW * (CHUNKS_PER_TILE + MAXPAD), CHUNK)

    deg = _sc_degree(dst2)
    dinv = lax.rsqrt(deg[0, :N_NODES] + deg[1, :N_NODES])[:, None]

    g1 = _mm_scale(x, W1, dinv)
    a1 = _sc_aggregate(g1, src3, dst2)
    g2 = _fused_layer(a1, g1, dinv, b1, W2)
    a2 = _sc_aggregate(g2, src3, dst2)
    g3 = _fused_layer(a2, g2, dinv, b2, W3)
    a3 = _sc_aggregate(g3, src3, dst2)
    h3 = _epilogue(a3, g3, dinv, b3)

    flat = h3.reshape(1, N_NODES * 32)
    return _final_matmul(flat, We, be)


# speedup vs baseline: 1.7322x; 1.7322x over previous
import functools

import jax
import jax.numpy as jnp
from jax import lax
from jax.experimental import pallas as pl
from jax.experimental.pallas import tpu as pltpu
from jax.experimental.pallas import tpu_sc as plsc

N_NODES = 10000
N_ACC = 10240
N_EDGES = 320000
NC, NS = 2, 16
CHUNK = 128
CHUNKS_PER_TILE = 80
E_PAD = NC * NS * CHUNKS_PER_TILE * CHUNK
E_ROWS = E_PAD // CHUNK
ROWS_PER_TILE = N_ACC // NS


def _sc_mesh():
    return plsc.VectorSubcoreMesh(core_axis_name="c", subcore_axis_name="s")


_SC_PARAMS = pltpu.CompilerParams(use_tc_tiling_on_sc=False)


def _degree_body(dst_hbm, out_hbm, dst_v, ones_v, deg_sh, sem):
    c = lax.axis_index("c")
    s = lax.axis_index("s")
    w = c * NS + s
    init = jnp.where(c == 0, 1.0, 0.0).astype(jnp.float32)

    for k in range(CHUNK // 16):
        ones_v[pl.ds(16 * k, 16)] = jnp.zeros((16,), jnp.float32) + init

    def zinit(t, _):
        pltpu.sync_copy(ones_v, deg_sh.at[pl.ds(ROWS_PER_TILE * s + CHUNK * t, CHUNK)])
        return 0

    lax.fori_loop(0, ROWS_PER_TILE // CHUNK, zinit, 0)
    for k in range(CHUNK // 16):
        ones_v[pl.ds(16 * k, 16)] = jnp.zeros((16,), jnp.float32) + 1.0
    plsc.subcore_barrier()

    pltpu.sync_copy(dst_hbm.at[pl.ds(w * CHUNKS_PER_TILE, CHUNKS_PER_TILE)], dst_v)

    def step(j, _):
        pltpu.sync_copy(ones_v, deg_sh.at[dst_v.at[j]], add=True)
        return 0

    lax.fori_loop(0, CHUNKS_PER_TILE, step, 0)
    plsc.subcore_barrier()
    pltpu.sync_copy(deg_sh.at[pl.ds(ROWS_PER_TILE * s, ROWS_PER_TILE)],
                    out_hbm.at[c, pl.ds(ROWS_PER_TILE * s, ROWS_PER_TILE)])


def _sc_degree(dst2):
    k = functools.partial(
        pl.kernel,
        out_type=jax.ShapeDtypeStruct((NC, N_ACC), jnp.float32),
        mesh=_sc_mesh(),
        scratch_types=[
            pltpu.VMEM((CHUNKS_PER_TILE, CHUNK), jnp.int32),
            pltpu.VMEM((CHUNK,), jnp.float32),
            pltpu.VMEM_SHARED((N_ACC,), jnp.float32),
            pltpu.SemaphoreType.DMA,
        ],
        compiler_params=_SC_PARAMS,
    )(_degree_body)
    return k(dst2)


ROWS_G_TILE = N_NODES // NS


def _make_agg_body(F, stage_g):
    def body(g_hbm, src_hbm, dst_hbm, out_hbm, *refs):
        if stage_g:
            src_v, dst_v, rows, g_sh, acc_sh, sem = refs
        else:
            src_v, dst_v, rows, acc_sh, sem = refs
            g_sh = None
        c = lax.axis_index("c")
        s = lax.axis_index("s")
        w = c * NS + s

        def zrow(r, _):
            for k in range(F // 16):
                rows[r, pl.ds(16 * k, 16)] = jnp.zeros((16,), jnp.float32)
            return 0

        lax.fori_loop(0, CHUNK, zrow, 0)

        def zinit(t, _):
            pltpu.sync_copy(
                rows, acc_sh.at[pl.ds(ROWS_PER_TILE * s + CHUNK * t, CHUNK)])
            return 0

        lax.fori_loop(0, ROWS_PER_TILE // CHUNK, zinit, 0)
        if stage_g:
            pltpu.sync_copy(g_hbm.at[pl.ds(ROWS_G_TILE * s, ROWS_G_TILE)],
                            g_sh.at[pl.ds(ROWS_G_TILE * s, ROWS_G_TILE)])
        plsc.subcore_barrier()

        pltpu.sync_copy(src_hbm.at[pl.ds(w * CHUNKS_PER_TILE, CHUNKS_PER_TILE)], src_v)
        pltpu.sync_copy(dst_hbm.at[pl.ds(w * CHUNKS_PER_TILE, CHUNKS_PER_TILE)], dst_v)

        g_src = g_sh if stage_g else g_hbm

        def step(j, _):
            pltpu.sync_copy(g_src.at[src_v.at[j]], rows)
            pltpu.sync_copy(rows, acc_sh.at[dst_v.at[j]], add=True)
            return 0

        lax.fori_loop(0, CHUNKS_PER_TILE, step, 0)
        plsc.subcore_barrier()
        pltpu.sync_copy(acc_sh.at[pl.ds(ROWS_PER_TILE * s, ROWS_PER_TILE)],
                        out_hbm.at[c, pl.ds(ROWS_PER_TILE * s, ROWS_PER_TILE)])

    return body


def _sc_aggregate(g, src2, dst2):
    F = g.shape[1]
    stage_g = F <= 64
    k = functools.partial(
        pl.kernel,
        out_type=jax.ShapeDtypeStruct((NC, N_ACC, F), jnp.float32),
        mesh=_sc_mesh(),
        scratch_types=(
            [pltpu.VMEM((CHUNKS_PER_TILE, CHUNK), jnp.int32),
             pltpu.VMEM((CHUNKS_PER_TILE, CHUNK), jnp.int32),
             pltpu.VMEM((CHUNK, F), jnp.float32)]
            + ([pltpu.VMEM_SHARED((N_NODES, F), jnp.float32)] if stage_g else [])
            + [pltpu.VMEM_SHARED((N_ACC, F), jnp.float32),
               pltpu.SemaphoreType.DMA]
        ),
        compiler_params=_SC_PARAMS,
    )(_make_agg_body(F, stage_g))
    return k(g, src2, dst2)


BN = 2000


def _mm_scale_body(x_ref, w_ref, dv_ref, g_ref):
    h = jnp.dot(x_ref[...], w_ref[...], preferred_element_type=jnp.float32)
    g_ref[...] = h * dv_ref[...]


def _mm_scale(x, W, dinv):
    Fin, Fout = W.shape
    grid = (N_NODES // BN,)
    return pl.pallas_call(
        _mm_scale_body,
        grid=grid,
        in_specs=[
            pl.BlockSpec((BN, Fin), lambda i: (i, 0)),
            pl.BlockSpec((Fin, Fout), lambda i: (0, 0)),
            pl.BlockSpec((BN, 1), lambda i: (i, 0)),
        ],
        out_specs=pl.BlockSpec((BN, Fout), lambda i: (i, 0)),
        out_shape=jax.ShapeDtypeStruct((N_NODES, Fout), jnp.float32),
    )(x, W, dinv)


def _fused_layer_body(acc_ref, g_ref, dv_ref, b_ref, w_ref, out_ref):
    dv = dv_ref[...]
    y = (acc_ref[0] + acc_ref[1] + g_ref[...]) * dv + b_ref[...]
    y = jnp.maximum(y, 0.0)
    out_ref[...] = jnp.dot(y, w_ref[...], preferred_element_type=jnp.float32) * dv


def _fused_layer(acc, g, dinv, b, W):
    Fin, Fout = W.shape
    grid = (N_NODES // BN,)
    return pl.pallas_call(
        _fused_layer_body,
        grid=grid,
        in_specs=[
            pl.BlockSpec((NC, BN, Fin), lambda i: (0, i, 0)),
            pl.BlockSpec((BN, Fin), lambda i: (i, 0)),
            pl.BlockSpec((BN, 1), lambda i: (i, 0)),
            pl.BlockSpec((1, Fin), lambda i: (0, 0)),
            pl.BlockSpec((Fin, Fout), lambda i: (0, 0)),
        ],
        out_specs=pl.BlockSpec((BN, Fout), lambda i: (i, 0)),
        out_shape=jax.ShapeDtypeStruct((N_NODES, Fout), jnp.float32),
    )(acc, g, dinv, b.reshape(1, Fin), W)


def _epilogue_body(acc_ref, g_ref, dv_ref, b_ref, out_ref):
    out_ref[...] = (acc_ref[0] + acc_ref[1] + g_ref[...]) * dv_ref[...] + b_ref[...]


def _epilogue(acc, g, dinv, b):
    F = g.shape[1]
    grid = (N_NODES // BN,)
    return pl.pallas_call(
        _epilogue_body,
        grid=grid,
        in_specs=[
            pl.BlockSpec((NC, BN, F), lambda i: (0, i, 0)),
            pl.BlockSpec((BN, F), lambda i: (i, 0)),
            pl.BlockSpec((BN, 1), lambda i: (i, 0)),
            pl.BlockSpec((1, F), lambda i: (0, 0)),
        ],
        out_specs=pl.BlockSpec((BN, F), lambda i: (i, 0)),
        out_shape=jax.ShapeDtypeStruct((N_NODES, F), jnp.float32),
    )(acc, g, dinv, b.reshape(1, F))


def _final_mm_body(flat_ref, we_ref, be_ref, z_ref):
    k = pl.program_id(0)

    @pl.when(k == 0)
    def _init():
        z_ref[...] = be_ref[...]

    z_ref[...] += jnp.dot(flat_ref[...], we_ref[...],
                          preferred_element_type=jnp.float32)


def _final_matmul(flat, We, be):
    K = flat.shape[1]
    BK = 16000
    grid = (K // BK,)
    return pl.pallas_call(
        _final_mm_body,
        grid=grid,
        in_specs=[
            pl.BlockSpec((1, BK), lambda k: (0, k)),
            pl.BlockSpec((BK, 128), lambda k: (k, 0)),
            pl.BlockSpec((1, 128), lambda k: (0, 0)),
        ],
        out_specs=pl.BlockSpec((1, 128), lambda k: (0, 0)),
        out_shape=jax.ShapeDtypeStruct((1, 128), jnp.float32),
    )(flat, We, be.reshape(1, 128))


def kernel(x, edge_index, W1, b1, W2, b2, W3, b3, We, be):
    src = edge_index[0].astype(jnp.int32)
    dst = edge_index[1].astype(jnp.int32)
    extra = E_PAD - N_EDGES
    pad_src = jnp.arange(extra, dtype=jnp.int32) % N_NODES
    pad_dst = N_NODES + jnp.arange(extra, dtype=jnp.int32) % (N_ACC - N_NODES)
    src2 = jnp.concatenate([src, pad_src]).reshape(E_ROWS, CHUNK)
    dst2 = jnp.concatenate([dst, pad_dst]).reshape(E_ROWS, CHUNK)

    deg = _sc_degree(dst2)
    dinv = lax.rsqrt(deg[0, :N_NODES] + deg[1, :N_NODES])[:, None]

    g1 = _mm_scale(x, W1, dinv)
    a1 = _sc_aggregate(g1, src2, dst2)
    g2 = _fused_layer(a1, g1, dinv, b1, W2)
    a2 = _sc_aggregate(g2, src2, dst2)
    g3 = _fused_layer(a2, g2, dinv, b2, W3)
    a3 = _sc_aggregate(g3, src2, dst2)
    h3 = _epilogue(a3, g3, dinv, b3)

    flat = h3.reshape(1, N_NODES * 32)
    return _final_matmul(flat, We, be)
